# Initial kernel scaffold; baseline (speedup 1.0000x reference)
import jax
import jax.numpy as jnp
from jax.experimental import pallas as pl


def kernel(speaker_labeles, table):
    raise NotImplementedError("write your pallas kernel here")


# speedup vs baseline: 3.0166x; 3.0166x over previous
import functools

import jax
import jax.numpy as jnp
from jax import lax
from jax.experimental import pallas as pl
from jax.experimental.pallas import tpu as pltpu
from jax.experimental.pallas import tpu_sc as plsc

_DIM = 32
_NC = 2
_NS = 16
_NW = _NC * _NS


@functools.lru_cache(maxsize=None)
def _build_gather(n, chunk):
    assert n % _NW == 0
    b_per_w = n // _NW
    assert b_per_w % chunk == 0 and chunk % 8 == 0
    nchunk = b_per_w // chunk
    mesh = plsc.VectorSubcoreMesh(core_axis_name="c", subcore_axis_name="s")

    @functools.partial(
        pl.kernel,
        out_type=jax.ShapeDtypeStruct((n, _DIM), jnp.float32),
        mesh=mesh,
        compiler_params=pltpu.CompilerParams(use_tc_tiling_on_sc=False),
        scratch_types=[
            pltpu.VMEM((b_per_w,), jnp.int32),
            pltpu.VMEM((chunk, _DIM), jnp.float32),
            pltpu.VMEM((chunk, _DIM), jnp.float32),
            pltpu.SemaphoreType.DMA,
            pltpu.SemaphoreType.DMA,
        ],
    )
    def grab(idx_hbm, table_hbm, out_hbm, idx_v, rows0, rows1, sem0, sem1):
        wid = lax.axis_index("s") * _NC + lax.axis_index("c")
        base = wid * b_per_w
        pltpu.sync_copy(idx_hbm.at[pl.ds(base, b_per_w)], idx_v)
        rows = (rows0, rows1)
        sems = (sem0, sem1)
        pending = [None, None]
        for g in range(min(2, nchunk)):
            pending[g] = pltpu.async_copy(
                table_hbm.at[idx_v.at[pl.ds(g * chunk, chunk)]], rows[g], sems[g])
        for g in range(nchunk):
            b = g % 2
            pending[b].wait()
            pltpu.sync_copy(rows[b], out_hbm.at[pl.ds(base + g * chunk, chunk)])
            nxt = g + 2
            if nxt < nchunk:
                pending[b] = pltpu.async_copy(
                    table_hbm.at[idx_v.at[pl.ds(nxt * chunk, chunk)]],
                    rows[b], sems[b])

    return grab


def kernel(speaker_labeles, table):
    batch, hist = speaker_labeles.shape
    idx = speaker_labeles.reshape(-1).astype(jnp.int32)
    out = _build_gather(batch * hist, 1600)(idx, table)
    return out.reshape(batch, hist, _DIM)


# speedup vs baseline: 6.1754x; 2.0471x over previous
import functools

import jax
import jax.numpy as jnp
from jax import lax
from jax.experimental import pallas as pl
from jax.experimental.pallas import tpu as pltpu
from jax.experimental.pallas import tpu_sc as plsc

_DIM = 32
_NC = 2
_NS = 16
_NW = _NC * _NS


@functools.lru_cache(maxsize=None)
def _build_gather(batch, hist, grp):
    assert batch % _NW == 0
    rows_per_w = batch // _NW
    assert rows_per_w % grp == 0
    ngrp = rows_per_w // grp
    assert ngrp % 2 == 0 and ngrp >= 4
    gsz = grp * hist
    mesh = plsc.VectorSubcoreMesh(core_axis_name="c", subcore_axis_name="s")

    @functools.partial(
        pl.kernel,
        out_type=jax.ShapeDtypeStruct((batch, hist, _DIM), jnp.float32),
        mesh=mesh,
        compiler_params=pltpu.CompilerParams(use_tc_tiling_on_sc=False),
        scratch_types=[
            pltpu.VMEM((rows_per_w, hist), jnp.int32),
            pltpu.VMEM((grp, hist, _DIM), jnp.float32),
            pltpu.VMEM((grp, hist, _DIM), jnp.float32),
            pltpu.SemaphoreType.DMA,
            pltpu.SemaphoreType.DMA,
        ],
    )
    def grab(idx_hbm, table_hbm, out_hbm, idx_v, rows0, rows1, sem0, sem1):
        wid = lax.axis_index("s") * _NC + lax.axis_index("c")
        row0 = wid * rows_per_w
        pltpu.sync_copy(idx_hbm.at[pl.ds(row0, rows_per_w)], idx_v)
        rows = (rows0, rows1)
        sems = (sem0, sem1)

        def fire(g, b):
            for r in range(grp):
                pltpu.async_copy(
                    table_hbm.at[idx_v.at[g * grp + r]],
                    rows[b].at[r],
                    sems[b],
                )

        def drain(b):
            pltpu.make_async_copy(
                out_hbm.at[pl.ds(0, grp)], rows[b], sems[b]
            ).wait()

        def flush(g, b):
            pltpu.sync_copy(
                rows[b], out_hbm.at[pl.ds(row0 + g * grp, grp)]
            )

        fire(0, 0)
        fire(1, 1)

        @pl.loop(0, ngrp - 2, step=2)
        def _steady(g):
            for b in (0, 1):
                drain(b)
                flush(g + b, b)
                fire(g + b + 2, b)

        for b in (0, 1):
            drain(b)
            flush(ngrp - 2 + b, b)

    return grab


def kernel(speaker_labeles, table):
    batch, hist = speaker_labeles.shape
    idx = speaker_labeles.astype(jnp.int32)
    return _build_gather(batch, hist, 8)(idx, table)
